# baseline (speedup 1.0000x reference)
import jax
import jax.numpy as jnp
from jax.experimental import pallas as pl
from jax.experimental.pallas import tpu as pltpu

_N_EVENTS = 10000
_NSPLITS = 5
_NF_IN = 4
_NF = 32
_NG = 16
_BE = 1000
_NNODES = 2 ** (_NSPLITS + 1) - 1
_C = 4


def _mm(x, w, b):
    return jnp.maximum(jnp.dot(x, w, preferred_element_type=jnp.float32) + b, 0.0)


def _bdiag_parts(ws, c):
    n = ws[0].shape[1]
    rows = []
    for w in ws:
        k = w.shape[0]
        blk = jnp.zeros((c * k, c * n), w.dtype)
        for j in range(c):
            blk = jax.lax.dynamic_update_slice(blk, w, (j * k, j * n))
        rows.append(blk)
    return jnp.concatenate(rows, axis=0)


def _btile(b, c):
    return jnp.tile(b.reshape(1, -1), (1, c))


def _dot(x, w):
    return jnp.dot(x, w, preferred_element_type=jnp.float32)


def _tree_body(rv_ref, *refs):
    out_ref = refs[-1]
    (pw0, pb0, pw1, pb1, qw0, qb0, qw1, qb1,
     bw0x, bw0g, bb0, bw1, bb1, bw2, bb2,
     mw0x, mw0g, mb0, mw1, mb1, mw2, mb2,
     uw0x, uw0a, uw0g, ub0, uw1, ub1, uw2, ub2) = [r[...] for r in refs[:-1]]
    be = rv_ref.shape[0]
    bc = be // _C
    rv = rv_ref[...]
    x0 = jnp.concatenate([rv[j * bc:(j + 1) * bc] for j in range(_C)], axis=1)
    xs = [x0.reshape(1, bc, _C * _NF)]
    for inx in range(_NSPLITS):
        na = 2 ** (inx + 1) - 1
        xa = jnp.concatenate([xs[l].reshape(-1, _C * _NF)
                              for l in range(inx + 1)], axis=0)
        h = _mm(_mm(xa, pw0, pb0), pw1, pb1)
        pooled = h.reshape(na, bc, _C * _NG).sum(axis=0) * (1.0 / na)
        g = _mm(_mm(pooled, qw0, qb0), qw1, qb1)
        nl = 2 ** inx
        leaf = xs[inx].reshape(nl * bc, _C * _NF)
        gcb = (_dot(g, bw0g) + bb0).reshape(1, bc, -1)
        cb0 = jnp.maximum(
            _dot(leaf, bw0x).reshape(nl, bc, -1) + gcb, 0.0
        ).reshape(nl * bc, -1)
        cb = _mm(_mm(cb0, bw1, bb1), bw2, bb2)
        c3 = cb.reshape(nl, bc, _C * 2 * _NF)
        c0 = jnp.concatenate(
            [c3[:, :, 2 * _NF * j:2 * _NF * j + _NF] for j in range(_C)],
            axis=2).reshape(nl, 1, bc, _C * _NF)
        c1 = jnp.concatenate(
            [c3[:, :, 2 * _NF * j + _NF:2 * _NF * (j + 1)] for j in range(_C)],
            axis=2).reshape(nl, 1, bc, _C * _NF)
        xs.append(jnp.concatenate([c0, c1], axis=1).reshape(
            2 * nl, bc, _C * _NF))
        gcm = (_dot(g, mw0g) + mb0).reshape(1, bc, -1)
        m0 = jnp.maximum(
            _dot(xa, mw0x).reshape(na, bc, -1) + gcm, 0.0
        ).reshape(na * bc, -1)
        m = _mm(_mm(m0, mw1, mb1), mw2, mb2)
        m = m.reshape(na, bc, _C * _NF)
        s_lvl = [m[0:1]]
        row = 1
        for l in range(1, inx + 1):
            cnt = 2 ** l
            s_lvl.append(jnp.repeat(s_lvl[l - 1], 2, axis=0) + m[row:row + cnt])
            row += cnt
        nn = 2 * na + 1
        x_all = jnp.concatenate([xs[l].reshape(-1, _C * _NF)
                                 for l in range(inx + 2)], axis=0)
        s_all = jnp.concatenate([s.reshape(-1, _C * _NF) for s in s_lvl],
                                axis=0)
        sw = _dot(s_all, uw0a).reshape(na, bc, -1)
        aggw = [jnp.zeros((1, bc, sw.shape[2]), jnp.float32)]
        row = 0
        for l in range(inx + 1):
            cnt = 2 ** l
            aggw.append(jnp.repeat(sw[row:row + cnt], 2, axis=0))
            row += cnt
        aggw_all = jnp.concatenate(aggw, axis=0)
        gcu = (_dot(g, uw0g) + ub0).reshape(1, bc, -1)
        u0 = jnp.maximum(
            _dot(x_all, uw0x).reshape(nn, bc, -1) + aggw_all + gcu, 0.0
        ).reshape(nn * bc, -1)
        u = _mm(_mm(u0, uw1, ub1), uw2, ub2)
        row = 0
        for l in range(inx + 2):
            cnt = 2 ** l
            xs[l] = u[row * bc:(row + cnt) * bc].reshape(cnt, bc, _C * _NF)
            row += cnt
    chunk_rows = []
    for j in range(_C):
        pieces = []
        for l in range(_NSPLITS + 1):
            for i in range(2 ** l):
                pieces.append(xs[l][i, :, _NF * j:_NF * j + _NF_IN])
        chunk_rows.append(jnp.concatenate(pieces, axis=1))
    out_ref[...] = jnp.concatenate(chunk_rows, axis=0)


def kernel(random_vector, hlvs_pre_w0, hlvs_pre_b0, hlvs_pre_w1, hlvs_pre_b1,
           hlvs_post_w0, hlvs_post_b0, hlvs_post_w1, hlvs_post_b1,
           br_w0, br_b0, br_w1, br_b1, br_w2, br_b2,
           msg_w0, msg_b0, msg_w1, msg_b1, msg_w2, msg_b2,
           upd_w0, upd_b0, upd_w1, upd_b1, upd_w2, upd_b2):
    c = _C
    weights = []
    for w, b in (
        (_bdiag_parts([hlvs_pre_w0], c), _btile(hlvs_pre_b0, c)),
        (_bdiag_parts([hlvs_pre_w1], c), _btile(hlvs_pre_b1, c)),
        (_bdiag_parts([hlvs_post_w0], c), _btile(hlvs_post_b0, c)),
        (_bdiag_parts([hlvs_post_w1], c), _btile(hlvs_post_b1, c)),
        (_bdiag_parts([br_w0[:_NF]], c), _bdiag_parts([br_w0[_NF:]], c)),
        (_btile(br_b0, c), _bdiag_parts([br_w1], c)),
        (_btile(br_b1, c), _bdiag_parts([br_w2], c)),
        (_btile(br_b2, c), _bdiag_parts([msg_w0[:_NF]], c)),
        (_bdiag_parts([msg_w0[_NF:]], c), _btile(msg_b0, c)),
        (_bdiag_parts([msg_w1], c), _btile(msg_b1, c)),
        (_bdiag_parts([msg_w2], c), _btile(msg_b2, c)),
        (_bdiag_parts([upd_w0[:_NF]], c), _bdiag_parts([upd_w0[_NF:2 * _NF]], c)),
        (_bdiag_parts([upd_w0[2 * _NF:]], c), _btile(upd_b0, c)),
        (_bdiag_parts([upd_w1], c), _btile(upd_b1, c)),
        (_bdiag_parts([upd_w2], c), _btile(upd_b2, c)),
    ):
        weights.append(w)
        weights.append(b)
    nblocks = _N_EVENTS // _BE
    wspecs = [pl.BlockSpec(w.shape, lambda i: (0, 0)) for w in weights]
    out = pl.pallas_call(
        _tree_body,
        grid=(nblocks,),
        in_specs=[pl.BlockSpec((_BE, _NF), lambda i: (i, 0))] + wspecs,
        out_specs=pl.BlockSpec((_BE, _NNODES * _NF_IN), lambda i: (i, 0)),
        out_shape=jax.ShapeDtypeStruct((_N_EVENTS, _NNODES * _NF_IN),
                                       jnp.float32),
        compiler_params=pltpu.CompilerParams(
            dimension_semantics=("parallel",)),
    )(random_vector, *weights)
    return out.reshape(_N_EVENTS, _NNODES, _NF_IN)


# speedup vs baseline: 1.5747x; 1.5747x over previous
import jax
import jax.numpy as jnp
from jax.experimental import pallas as pl
from jax.experimental.pallas import tpu as pltpu

_N_EVENTS = 10000
_NSPLITS = 5
_NF_IN = 4
_NF = 32
_NG = 16
_BE = 400
_NNODES = 2 ** (_NSPLITS + 1) - 1
_C = 4


def _mm(x, w, b):
    return jnp.maximum(jnp.dot(x, w, preferred_element_type=jnp.float32) + b, 0.0)


def _bdiag_parts(ws, c):
    n = ws[0].shape[1]
    rows = []
    for w in ws:
        k = w.shape[0]
        blk = jnp.zeros((c * k, c * n), w.dtype)
        for j in range(c):
            blk = jax.lax.dynamic_update_slice(blk, w, (j * k, j * n))
        rows.append(blk)
    return jnp.concatenate(rows, axis=0)


def _btile(b, c):
    return jnp.tile(b.reshape(1, -1), (1, c))


def _dot(x, w):
    return jnp.dot(x, w, preferred_element_type=jnp.float32)


def _tree_body(rv_ref, *refs):
    out_ref = refs[-1]
    (pw0, pb0, pw1, pb1, qw0, qb0, qw1, qb1,
     bw0x, bw0g, bb0, bw1, bb1, bw2, bb2,
     mw0x, mw0g, mb0, mw1, mb1, mw2, mb2,
     uw0x, uw0a, uw0g, ub0, uw1, ub1, uw2, ub2) = [r[...] for r in refs[:-1]]
    be = rv_ref.shape[0]
    bc = be // _C
    rv = rv_ref[...]
    x0 = jnp.concatenate([rv[j * bc:(j + 1) * bc] for j in range(_C)], axis=1)
    xs = [x0.reshape(1, bc, _C * _NF)]
    for inx in range(_NSPLITS):
        na = 2 ** (inx + 1) - 1
        xa = jnp.concatenate([xs[l].reshape(-1, _C * _NF)
                              for l in range(inx + 1)], axis=0)
        h = _mm(_mm(xa, pw0, pb0), pw1, pb1)
        pooled = h.reshape(na, bc, _C * _NG).sum(axis=0) * (1.0 / na)
        g = _mm(_mm(pooled, qw0, qb0), qw1, qb1)
        nl = 2 ** inx
        leaf = xs[inx].reshape(nl * bc, _C * _NF)
        gcb = (_dot(g, bw0g) + bb0).reshape(1, bc, -1)
        cb0 = jnp.maximum(
            _dot(leaf, bw0x).reshape(nl, bc, -1) + gcb, 0.0
        ).reshape(nl * bc, -1)
        cb = _mm(_mm(cb0, bw1, bb1), bw2, bb2)
        c3 = cb.reshape(nl, bc, _C * 2 * _NF)
        c0 = jnp.concatenate(
            [c3[:, :, 2 * _NF * j:2 * _NF * j + _NF] for j in range(_C)],
            axis=2).reshape(nl, 1, bc, _C * _NF)
        c1 = jnp.concatenate(
            [c3[:, :, 2 * _NF * j + _NF:2 * _NF * (j + 1)] for j in range(_C)],
            axis=2).reshape(nl, 1, bc, _C * _NF)
        xs.append(jnp.concatenate([c0, c1], axis=1).reshape(
            2 * nl, bc, _C * _NF))
        gcm = (_dot(g, mw0g) + mb0).reshape(1, bc, -1)
        m0 = jnp.maximum(
            _dot(xa, mw0x).reshape(na, bc, -1) + gcm, 0.0
        ).reshape(na * bc, -1)
        m = _mm(_mm(m0, mw1, mb1), mw2, mb2)
        m = m.reshape(na, bc, _C * _NF)
        s_lvl = [m[0:1]]
        row = 1
        for l in range(1, inx + 1):
            cnt = 2 ** l
            s_lvl.append(jnp.repeat(s_lvl[l - 1], 2, axis=0) + m[row:row + cnt])
            row += cnt
        nn = 2 * na + 1
        x_all = jnp.concatenate([xs[l].reshape(-1, _C * _NF)
                                 for l in range(inx + 2)], axis=0)
        s_all = jnp.concatenate([s.reshape(-1, _C * _NF) for s in s_lvl],
                                axis=0)
        sw = _dot(s_all, uw0a).reshape(na, bc, -1)
        aggw = [jnp.zeros((1, bc, sw.shape[2]), jnp.float32)]
        row = 0
        for l in range(inx + 1):
            cnt = 2 ** l
            aggw.append(jnp.repeat(sw[row:row + cnt], 2, axis=0))
            row += cnt
        aggw_all = jnp.concatenate(aggw, axis=0)
        gcu = (_dot(g, uw0g) + ub0).reshape(1, bc, -1)
        u0 = jnp.maximum(
            _dot(x_all, uw0x).reshape(nn, bc, -1) + aggw_all + gcu, 0.0
        ).reshape(nn * bc, -1)
        u = _mm(_mm(u0, uw1, ub1), uw2, ub2)
        row = 0
        for l in range(inx + 2):
            cnt = 2 ** l
            xs[l] = u[row * bc:(row + cnt) * bc].reshape(cnt, bc, _C * _NF)
            row += cnt
    chunk_rows = []
    for j in range(_C):
        pieces = []
        for l in range(_NSPLITS + 1):
            for i in range(2 ** l):
                pieces.append(xs[l][i, :, _NF * j:_NF * j + _NF_IN])
        chunk_rows.append(jnp.concatenate(pieces, axis=1))
    out_ref[...] = jnp.concatenate(chunk_rows, axis=0)


def kernel(random_vector, hlvs_pre_w0, hlvs_pre_b0, hlvs_pre_w1, hlvs_pre_b1,
           hlvs_post_w0, hlvs_post_b0, hlvs_post_w1, hlvs_post_b1,
           br_w0, br_b0, br_w1, br_b1, br_w2, br_b2,
           msg_w0, msg_b0, msg_w1, msg_b1, msg_w2, msg_b2,
           upd_w0, upd_b0, upd_w1, upd_b1, upd_w2, upd_b2):
    c = _C
    weights = []
    for w, b in (
        (_bdiag_parts([hlvs_pre_w0], c), _btile(hlvs_pre_b0, c)),
        (_bdiag_parts([hlvs_pre_w1], c), _btile(hlvs_pre_b1, c)),
        (_bdiag_parts([hlvs_post_w0], c), _btile(hlvs_post_b0, c)),
        (_bdiag_parts([hlvs_post_w1], c), _btile(hlvs_post_b1, c)),
        (_bdiag_parts([br_w0[:_NF]], c), _bdiag_parts([br_w0[_NF:]], c)),
        (_btile(br_b0, c), _bdiag_parts([br_w1], c)),
        (_btile(br_b1, c), _bdiag_parts([br_w2], c)),
        (_btile(br_b2, c), _bdiag_parts([msg_w0[:_NF]], c)),
        (_bdiag_parts([msg_w0[_NF:]], c), _btile(msg_b0, c)),
        (_bdiag_parts([msg_w1], c), _btile(msg_b1, c)),
        (_bdiag_parts([msg_w2], c), _btile(msg_b2, c)),
        (_bdiag_parts([upd_w0[:_NF]], c), _bdiag_parts([upd_w0[_NF:2 * _NF]], c)),
        (_bdiag_parts([upd_w0[2 * _NF:]], c), _btile(upd_b0, c)),
        (_bdiag_parts([upd_w1], c), _btile(upd_b1, c)),
        (_bdiag_parts([upd_w2], c), _btile(upd_b2, c)),
    ):
        weights.append(w)
        weights.append(b)
    nblocks = _N_EVENTS // _BE
    wspecs = [pl.BlockSpec(w.shape, lambda i: (0, 0)) for w in weights]
    out = pl.pallas_call(
        _tree_body,
        grid=(nblocks,),
        in_specs=[pl.BlockSpec((_BE, _NF), lambda i: (i, 0))] + wspecs,
        out_specs=pl.BlockSpec((_BE, _NNODES * _NF_IN), lambda i: (i, 0)),
        out_shape=jax.ShapeDtypeStruct((_N_EVENTS, _NNODES * _NF_IN),
                                       jnp.float32),
        compiler_params=pltpu.CompilerParams(
            dimension_semantics=("parallel",)),
    )(random_vector, *weights)
    return out.reshape(_N_EVENTS, _NNODES, _NF_IN)
